# baseline (speedup 1.0000x reference)
import dataclasses
import functools

import jax
import jax.numpy as jnp
from jax import lax
from jax.experimental import pallas as pl
from jax.experimental.pallas import tpu as pltpu
from jax.experimental.pallas import tpu_sc as plsc

N = 10000
E = 320000
D = 128
R = 8

NC = 2
NS = 16
NW = NC * NS
CHUNK = 128
JCHUNKS = 80
EPW = CHUNK * JCHUNKS
EP = EPW * NW
NCHUNKS = EP // CHUNK
REC = 3 * CHUNK

S_FULL = 640
S_LAST = N - (NS - 1) * S_FULL

BN = 400
BP = 320
BC = 2000


def _mm_body(f_ref, w_ref, o_ref):
    o_ref[0] = jnp.dot(f_ref[...], w_ref[0], preferred_element_type=jnp.float32)


def _make_table(feature, weight):
    return pl.pallas_call(
        _mm_body,
        grid=(R, N // BN),
        in_specs=[
            pl.BlockSpec((BN, D), lambda r, i: (i, 0)),
            pl.BlockSpec((1, D, D), lambda r, i: (r, 0, 0)),
        ],
        out_specs=pl.BlockSpec((1, BN, D), lambda r, i: (r, i, 0)),
        out_shape=jax.ShapeDtypeStruct((R, N, D), jnp.float32),
    )(feature, weight)


def _prep_body(s_ref, e_ref, d_ref, n_ref, o_ref):
    o_ref[:, 0, :] = e_ref[...] * N + s_ref[...]
    o_ref[:, 1, :] = d_ref[...]
    o_ref[:, 2, :] = lax.bitcast_convert_type(n_ref[...], jnp.int32)


def _pack_edges(srcp, dstp, etp, normp):
    s2 = srcp.reshape(NCHUNKS, CHUNK)
    e2 = etp.reshape(NCHUNKS, CHUNK)
    d2 = dstp.reshape(NCHUNKS, CHUNK)
    n2 = normp.reshape(NCHUNKS, CHUNK)
    packed = pl.pallas_call(
        _prep_body,
        grid=(NCHUNKS // BP,),
        in_specs=[pl.BlockSpec((BP, CHUNK), lambda i: (i, 0))] * 4,
        out_specs=pl.BlockSpec((BP, 3, CHUNK), lambda i: (i, 0, 0)),
        out_shape=jax.ShapeDtypeStruct((NCHUNKS, 3, CHUNK), jnp.int32),
    )(s2, e2, d2, n2)
    return packed.reshape(NCHUNKS * REC)


def _sc_compiler_params():
    cp = pltpu.CompilerParams()
    if "needs_layout_passes" in pltpu.CompilerParams.__dataclass_fields__:
        cp = dataclasses.replace(cp, needs_layout_passes=False)
    return cp


def _sc_scatter(table, packed, zeros, half):
    mesh = plsc.VectorSubcoreMesh(core_axis_name="c", subcore_axis_name="s",
                                  num_cores=1)

    @functools.partial(
        pl.kernel,
        compiler_params=_sc_compiler_params(),
        out_type=jax.ShapeDtypeStruct((N, D), jnp.float32),
        mesh=mesh,
        scratch_types=[
            pltpu.VMEM((REC,), jnp.int32),
            pltpu.VMEM((REC,), jnp.int32),
            pltpu.VMEM((REC,), jnp.int32),
            pltpu.VMEM((REC,), jnp.int32),
            pltpu.VMEM((CHUNK,), jnp.int32),
            pltpu.VMEM((CHUNK, D), jnp.float32),
            pltpu.VMEM((CHUNK, D), jnp.float32),
            pltpu.VMEM_SHARED((N, D), jnp.float32),
            pltpu.SemaphoreType.DMA,
            pltpu.SemaphoreType.DMA,
            pltpu.SemaphoreType.DMA,
            pltpu.SemaphoreType.DMA,
            pltpu.SemaphoreType.DMA,
            pltpu.SemaphoreType.DMA,
        ],
    )
    def k(table_h, packed_h, zeros_h, out_h,
          pb0, pb1, pb2, pb3, db, rb0, rb1, acc,
          sp0, sp1, sp2, sp3, sg0, sg1):
        sid = lax.axis_index("s")
        wid = half * NS + sid
        r0 = pl.multiple_of(sid * S_FULL, 128)

        @pl.when(sid < NS - 1)
        def _():
            pltpu.sync_copy(zeros_h.at[pl.ds(r0, S_FULL)],
                            acc.at[pl.ds(r0, S_FULL)])

        @pl.when(sid == NS - 1)
        def _():
            pltpu.sync_copy(zeros_h.at[pl.ds(r0, S_LAST)],
                            acc.at[pl.ds(r0, S_LAST)])

        plsc.subcore_barrier()

        def rec_off(j):
            return pl.multiple_of((wid * JCHUNKS + j) * REC, 128)

        def fire_pk(j, pb, sp):
            pltpu.make_async_copy(
                packed_h.at[pl.ds(rec_off(j), REC)], pb, sp).start()

        def wait_pk(j, pb, sp):
            pltpu.make_async_copy(
                packed_h.at[pl.ds(rec_off(j), REC)], pb, sp).wait()

        def fire_g(pb, rb, sg):
            pltpu.make_async_copy(
                table_h.at[pb.at[pl.ds(0, CHUNK)]], rb, sg).start()

        def wait_g(pb, rb, sg):
            pltpu.make_async_copy(
                table_h.at[pb.at[pl.ds(0, CHUNK)]], rb, sg).wait()

        def process(pb, rb):
            for m in range(CHUNK // 16):
                db[pl.ds(m * 16, 16)] = pb[pl.ds(CHUNK + m * 16, 16)]

            @pl.loop(0, CHUNK, step=16)
            def _(i):
                nv16 = plsc.bitcast(pb[pl.ds(2 * CHUNK + i, 16)], jnp.float32)
                for l in range(16):
                    s = nv16[l]
                    for c in range(D // 16):
                        rb[i + l, pl.ds(c * 16, 16)] = (
                            rb[i + l, pl.ds(c * 16, 16)] * s)

            pltpu.sync_copy(rb, acc.at[db], add=True)

        def slot(jj, pb_cur, sp_cur, pb_n2, sp_n2, rb, sg):
            wait_g(pb_cur, rb, sg)
            process(pb_cur, rb)

            @pl.when(jj + 4 < JCHUNKS)
            def _():
                fire_pk(jj + 4, pb_cur, sp_cur)

            @pl.when(jj + 2 < JCHUNKS)
            def _():
                wait_pk(jj + 2, pb_n2, sp_n2)
                fire_g(pb_n2, rb, sg)

        fire_pk(0, pb0, sp0)
        fire_pk(1, pb1, sp1)
        fire_pk(2, pb2, sp2)
        fire_pk(3, pb3, sp3)
        wait_pk(0, pb0, sp0)
        fire_g(pb0, rb0, sg0)
        wait_pk(1, pb1, sp1)
        fire_g(pb1, rb1, sg1)

        @pl.loop(0, JCHUNKS, step=4)
        def _(j):
            slot(j + 0, pb0, sp0, pb2, sp2, rb0, sg0)
            slot(j + 1, pb1, sp1, pb3, sp3, rb1, sg1)
            slot(j + 2, pb2, sp2, pb0, sp0, rb0, sg0)
            slot(j + 3, pb3, sp3, pb1, sp1, rb1, sg1)

        plsc.subcore_barrier()

        @pl.when(sid < NS - 1)
        def _():
            pltpu.sync_copy(acc.at[pl.ds(r0, S_FULL)],
                            out_h.at[pl.ds(r0, S_FULL)])

        @pl.when(sid == NS - 1)
        def _():
            pltpu.sync_copy(acc.at[pl.ds(r0, S_LAST)],
                            out_h.at[pl.ds(r0, S_LAST)])

    return k(table, packed, zeros)


def _combine_body(a_ref, b_ref, o_ref):
    o_ref[...] = a_ref[...] + b_ref[...]


def _combine(a, b):
    return pl.pallas_call(
        _combine_body,
        grid=(N // BC,),
        in_specs=[pl.BlockSpec((BC, D), lambda i: (i, 0))] * 2,
        out_specs=pl.BlockSpec((BC, D), lambda i: (i, 0)),
        out_shape=jax.ShapeDtypeStruct((N, D), jnp.float32),
    )(a, b)


def kernel(feature, edge_index, edge_type, norm, weight):
    src = edge_index[0]
    dst = edge_index[1]
    nrm = norm[:, 0]
    pad = EP - E
    zi = jnp.zeros((pad,), jnp.int32)
    zf = jnp.zeros((pad,), jnp.float32)
    srcp = jnp.concatenate([src, zi])
    etp = jnp.concatenate([edge_type, zi])
    dstp = jnp.concatenate([dst, zi])
    normp = jnp.concatenate([nrm, zf])
    packed = _pack_edges(srcp, dstp, etp, normp)
    table = _make_table(feature, weight).reshape(R * N, D)
    zeros = jnp.zeros((N, D), jnp.float32)
    partial0 = _sc_scatter(table, packed, zeros, 0)
    partial1 = _sc_scatter(table, packed, zeros, 1)
    return _combine(partial0, partial1)


# speedup vs baseline: 2.5478x; 2.5478x over previous
import dataclasses
import functools

import jax
import jax.numpy as jnp
from jax import lax
from jax.experimental import pallas as pl
from jax.experimental.pallas import tpu as pltpu
from jax.experimental.pallas import tpu_sc as plsc

N = 10000
E = 320000
D = 128
R = 8

NC = 2
NS = 16
NW = NC * NS
CHUNK = 128
JCHUNKS = 80
EPW = CHUNK * JCHUNKS
EP = EPW * NW
NCHUNKS = EP // CHUNK
REC = 3 * CHUNK

S_FULL = 640
S_LAST = N - (NS - 1) * S_FULL

BN = 400
BP = 320
BC = 2000


def _mm_body(f_ref, w_ref, o_ref):
    o_ref[0] = jnp.dot(f_ref[...], w_ref[0], preferred_element_type=jnp.float32)


def _make_table(feature, weight):
    return pl.pallas_call(
        _mm_body,
        grid=(R, N // BN),
        in_specs=[
            pl.BlockSpec((BN, D), lambda r, i: (i, 0)),
            pl.BlockSpec((1, D, D), lambda r, i: (r, 0, 0)),
        ],
        out_specs=pl.BlockSpec((1, BN, D), lambda r, i: (r, i, 0)),
        out_shape=jax.ShapeDtypeStruct((R, N, D), jnp.float32),
    )(feature, weight)


def _prep_body(s_ref, e_ref, d_ref, n_ref, o_ref):
    o_ref[:, 0, :] = e_ref[...] * N + s_ref[...]
    o_ref[:, 1, :] = d_ref[...]
    o_ref[:, 2, :] = lax.bitcast_convert_type(n_ref[...], jnp.int32)


def _pack_edges(srcp, dstp, etp, normp):
    s2 = srcp.reshape(NCHUNKS, CHUNK)
    e2 = etp.reshape(NCHUNKS, CHUNK)
    d2 = dstp.reshape(NCHUNKS, CHUNK)
    n2 = normp.reshape(NCHUNKS, CHUNK)
    packed = pl.pallas_call(
        _prep_body,
        grid=(NCHUNKS // BP,),
        in_specs=[pl.BlockSpec((BP, CHUNK), lambda i: (i, 0))] * 4,
        out_specs=pl.BlockSpec((BP, 3, CHUNK), lambda i: (i, 0, 0)),
        out_shape=jax.ShapeDtypeStruct((NCHUNKS, 3, CHUNK), jnp.int32),
    )(s2, e2, d2, n2)
    return packed.reshape(NCHUNKS * REC)


def _sc_compiler_params():
    cp = pltpu.CompilerParams()
    if "needs_layout_passes" in pltpu.CompilerParams.__dataclass_fields__:
        cp = dataclasses.replace(cp, needs_layout_passes=False)
    return cp


def _sc_scatter(table, packed, zeros):
    mesh = plsc.VectorSubcoreMesh(core_axis_name="c", subcore_axis_name="s")

    @functools.partial(
        pl.kernel,
        compiler_params=_sc_compiler_params(),
        out_type=jax.ShapeDtypeStruct((NC, N, D), jnp.float32),
        mesh=mesh,
        scratch_types=[
            pltpu.VMEM((REC,), jnp.int32),
            pltpu.VMEM((REC,), jnp.int32),
            pltpu.VMEM((REC,), jnp.int32),
            pltpu.VMEM((REC,), jnp.int32),
            pltpu.VMEM((CHUNK,), jnp.int32),
            pltpu.VMEM((CHUNK, D), jnp.float32),
            pltpu.VMEM((CHUNK, D), jnp.float32),
            pltpu.VMEM_SHARED((N, D), jnp.float32),
            pltpu.SemaphoreType.DMA,
            pltpu.SemaphoreType.DMA,
            pltpu.SemaphoreType.DMA,
            pltpu.SemaphoreType.DMA,
            pltpu.SemaphoreType.DMA,
            pltpu.SemaphoreType.DMA,
        ],
    )
    def k(table_h, packed_h, zeros_h, out_h,
          pb0, pb1, pb2, pb3, db, rb0, rb1, acc,
          sp0, sp1, sp2, sp3, sg0, sg1):
        cid = lax.axis_index("c")
        sid = lax.axis_index("s")
        wid = cid * NS + sid
        r0 = pl.multiple_of(sid * S_FULL, 128)

        @pl.when(sid < NS - 1)
        def _():
            pltpu.sync_copy(zeros_h.at[pl.ds(r0, S_FULL)],
                            acc.at[pl.ds(r0, S_FULL)])

        @pl.when(sid == NS - 1)
        def _():
            pltpu.sync_copy(zeros_h.at[pl.ds(r0, S_LAST)],
                            acc.at[pl.ds(r0, S_LAST)])

        plsc.subcore_barrier()

        def rec_off(j):
            return pl.multiple_of((wid * JCHUNKS + j) * REC, 128)

        def fire_pk(j, pb, sp):
            pltpu.make_async_copy(
                packed_h.at[pl.ds(rec_off(j), REC)], pb, sp).start()

        def wait_pk(j, pb, sp):
            pltpu.make_async_copy(
                packed_h.at[pl.ds(rec_off(j), REC)], pb, sp).wait()

        def fire_g(pb, rb, sg):
            pltpu.make_async_copy(
                table_h.at[pb.at[pl.ds(0, CHUNK)]], rb, sg).start()

        def wait_g(pb, rb, sg):
            pltpu.make_async_copy(
                table_h.at[pb.at[pl.ds(0, CHUNK)]], rb, sg).wait()

        def process(pb, rb):
            for m in range(CHUNK // 16):
                db[pl.ds(m * 16, 16)] = pb[pl.ds(CHUNK + m * 16, 16)]

            @pl.loop(0, CHUNK, step=16)
            def _(i):
                nv16 = plsc.bitcast(pb[pl.ds(2 * CHUNK + i, 16)], jnp.float32)
                for l in range(16):
                    s = nv16[l]
                    for c in range(D // 16):
                        rb[i + l, pl.ds(c * 16, 16)] = (
                            rb[i + l, pl.ds(c * 16, 16)] * s)

            pltpu.sync_copy(rb, acc.at[db], add=True)

        def slot(jj, pb_cur, sp_cur, pb_n2, sp_n2, rb, sg):
            wait_g(pb_cur, rb, sg)
            process(pb_cur, rb)

            @pl.when(jj + 4 < JCHUNKS)
            def _():
                fire_pk(jj + 4, pb_cur, sp_cur)

            @pl.when(jj + 2 < JCHUNKS)
            def _():
                wait_pk(jj + 2, pb_n2, sp_n2)
                fire_g(pb_n2, rb, sg)

        fire_pk(0, pb0, sp0)
        fire_pk(1, pb1, sp1)
        fire_pk(2, pb2, sp2)
        fire_pk(3, pb3, sp3)
        wait_pk(0, pb0, sp0)
        fire_g(pb0, rb0, sg0)
        wait_pk(1, pb1, sp1)
        fire_g(pb1, rb1, sg1)

        @pl.loop(0, JCHUNKS, step=4)
        def _(j):
            slot(j + 0, pb0, sp0, pb2, sp2, rb0, sg0)
            slot(j + 1, pb1, sp1, pb3, sp3, rb1, sg1)
            slot(j + 2, pb2, sp2, pb0, sp0, rb0, sg0)
            slot(j + 3, pb3, sp3, pb1, sp1, rb1, sg1)

        plsc.subcore_barrier()

        @pl.when(sid < NS - 1)
        def _():
            pltpu.sync_copy(acc.at[pl.ds(r0, S_FULL)],
                            out_h.at[cid].at[pl.ds(r0, S_FULL)])

        @pl.when(sid == NS - 1)
        def _():
            pltpu.sync_copy(acc.at[pl.ds(r0, S_LAST)],
                            out_h.at[cid].at[pl.ds(r0, S_LAST)])

    return k(table, packed, zeros)


def _combine_body(p_ref, o_ref):
    o_ref[...] = p_ref[0] + p_ref[1]


def _combine(partial):
    return pl.pallas_call(
        _combine_body,
        grid=(N // BC,),
        in_specs=[pl.BlockSpec((NC, BC, D), lambda i: (0, i, 0))],
        out_specs=pl.BlockSpec((BC, D), lambda i: (i, 0)),
        out_shape=jax.ShapeDtypeStruct((N, D), jnp.float32),
    )(partial)


def kernel(feature, edge_index, edge_type, norm, weight):
    src = edge_index[0]
    dst = edge_index[1]
    nrm = norm[:, 0]
    pad = EP - E
    spread = jnp.arange(pad, dtype=jnp.int32) % N
    zf = jnp.zeros((pad,), jnp.float32)
    srcp = jnp.concatenate([src, spread])
    etp = jnp.concatenate([edge_type,
                           jnp.arange(pad, dtype=jnp.int32) % R])
    dstp = jnp.concatenate([dst, spread])
    normp = jnp.concatenate([nrm, zf])
    packed = _pack_edges(srcp, dstp, etp, normp)
    table = _make_table(feature, weight).reshape(R * N, D)
    zeros = jnp.zeros((N, D), jnp.float32)
    partial = _sc_scatter(table, packed, zeros)
    return _combine(partial)


# speedup vs baseline: 2.5692x; 1.0084x over previous
import dataclasses
import functools

import jax
import jax.numpy as jnp
from jax import lax
from jax.experimental import pallas as pl
from jax.experimental.pallas import tpu as pltpu
from jax.experimental.pallas import tpu_sc as plsc

N = 10000
E = 320000
D = 128
R = 8

NC = 2
NS = 16
NW = NC * NS
CHUNK = 128
JCHUNKS = 80
EPW = CHUNK * JCHUNKS
EP = EPW * NW
NCHUNKS = EP // CHUNK
REC = 3 * CHUNK

S_FULL = 640
S_LAST = N - (NS - 1) * S_FULL

BN = 400
BP = 320
BC = 2000


def _mm_body(f_ref, w_ref, o_ref):
    o_ref[0] = jnp.dot(f_ref[...], w_ref[0], preferred_element_type=jnp.float32)


def _make_table(feature, weight):
    return pl.pallas_call(
        _mm_body,
        grid=(R, N // BN),
        in_specs=[
            pl.BlockSpec((BN, D), lambda r, i: (i, 0)),
            pl.BlockSpec((1, D, D), lambda r, i: (r, 0, 0)),
        ],
        out_specs=pl.BlockSpec((1, BN, D), lambda r, i: (r, i, 0)),
        out_shape=jax.ShapeDtypeStruct((R, N, D), jnp.float32),
    )(feature, weight)


EC = E // CHUNK


def _prep_body(s_ref, e_ref, d_ref, n_ref, o_ref):
    o_ref[0:EC, 0, :] = e_ref[...] * N + s_ref[...]
    o_ref[0:EC, 1, :] = d_ref[...]
    o_ref[0:EC, 2, :] = lax.bitcast_convert_type(n_ref[...], jnp.int32)
    q = lax.broadcasted_iota(jnp.int32, (NCHUNKS - EC, CHUNK), 0)
    l = lax.broadcasted_iota(jnp.int32, (NCHUNKS - EC, CHUNK), 1)
    p = q * CHUNK + l
    o_ref[EC:NCHUNKS, 0, :] = (p % R) * N + p
    o_ref[EC:NCHUNKS, 1, :] = p
    o_ref[EC:NCHUNKS, 2, :] = jnp.zeros((NCHUNKS - EC, CHUNK), jnp.int32)


def _pack_edges(src, dst, et, nrm):
    packed = pl.pallas_call(
        _prep_body,
        out_shape=jax.ShapeDtypeStruct((NCHUNKS, 3, CHUNK), jnp.int32),
    )(src.reshape(EC, CHUNK), et.reshape(EC, CHUNK),
      dst.reshape(EC, CHUNK), nrm.reshape(EC, CHUNK))
    return packed.reshape(NCHUNKS * REC)


def _sc_compiler_params():
    cp = pltpu.CompilerParams()
    if "needs_layout_passes" in pltpu.CompilerParams.__dataclass_fields__:
        cp = dataclasses.replace(cp, needs_layout_passes=False)
    return cp


def _sc_scatter(table, packed, zeros):
    mesh = plsc.VectorSubcoreMesh(core_axis_name="c", subcore_axis_name="s")

    @functools.partial(
        pl.kernel,
        compiler_params=_sc_compiler_params(),
        out_type=jax.ShapeDtypeStruct((NC, N, D), jnp.float32),
        mesh=mesh,
        scratch_types=[
            pltpu.VMEM((REC,), jnp.int32),
            pltpu.VMEM((REC,), jnp.int32),
            pltpu.VMEM((REC,), jnp.int32),
            pltpu.VMEM((REC,), jnp.int32),
            pltpu.VMEM((CHUNK,), jnp.int32),
            pltpu.VMEM((CHUNK, D), jnp.float32),
            pltpu.VMEM((CHUNK, D), jnp.float32),
            pltpu.VMEM_SHARED((N, D), jnp.float32),
            pltpu.SemaphoreType.DMA,
            pltpu.SemaphoreType.DMA,
            pltpu.SemaphoreType.DMA,
            pltpu.SemaphoreType.DMA,
            pltpu.SemaphoreType.DMA,
            pltpu.SemaphoreType.DMA,
        ],
    )
    def k(table_h, packed_h, zeros_h, out_h,
          pb0, pb1, pb2, pb3, db, rb0, rb1, acc,
          sp0, sp1, sp2, sp3, sg0, sg1):
        cid = lax.axis_index("c")
        sid = lax.axis_index("s")
        wid = cid * NS + sid
        r0 = pl.multiple_of(sid * S_FULL, 128)

        @pl.when(sid < NS - 1)
        def _():
            pltpu.sync_copy(zeros_h.at[pl.ds(r0, S_FULL)],
                            acc.at[pl.ds(r0, S_FULL)])

        @pl.when(sid == NS - 1)
        def _():
            pltpu.sync_copy(zeros_h.at[pl.ds(r0, S_LAST)],
                            acc.at[pl.ds(r0, S_LAST)])

        plsc.subcore_barrier()

        def rec_off(j):
            return pl.multiple_of((wid * JCHUNKS + j) * REC, 128)

        def fire_pk(j, pb, sp):
            pltpu.make_async_copy(
                packed_h.at[pl.ds(rec_off(j), REC)], pb, sp).start()

        def wait_pk(j, pb, sp):
            pltpu.make_async_copy(
                packed_h.at[pl.ds(rec_off(j), REC)], pb, sp).wait()

        def fire_g(pb, rb, sg):
            pltpu.make_async_copy(
                table_h.at[pb.at[pl.ds(0, CHUNK)]], rb, sg).start()

        def wait_g(pb, rb, sg):
            pltpu.make_async_copy(
                table_h.at[pb.at[pl.ds(0, CHUNK)]], rb, sg).wait()

        def process(pb, rb):
            for m in range(CHUNK // 16):
                db[pl.ds(m * 16, 16)] = pb[pl.ds(CHUNK + m * 16, 16)]

            @pl.loop(0, CHUNK, step=16)
            def _(i):
                nv16 = plsc.bitcast(pb[pl.ds(2 * CHUNK + i, 16)], jnp.float32)
                for l in range(16):
                    s = nv16[l]
                    for c in range(D // 16):
                        rb[i + l, pl.ds(c * 16, 16)] = (
                            rb[i + l, pl.ds(c * 16, 16)] * s)

            pltpu.sync_copy(rb, acc.at[db], add=True)

        def slot(jj, pb_cur, sp_cur, pb_n2, sp_n2, rb, sg):
            wait_g(pb_cur, rb, sg)
            process(pb_cur, rb)

            @pl.when(jj + 4 < JCHUNKS)
            def _():
                fire_pk(jj + 4, pb_cur, sp_cur)

            @pl.when(jj + 2 < JCHUNKS)
            def _():
                wait_pk(jj + 2, pb_n2, sp_n2)
                fire_g(pb_n2, rb, sg)

        fire_pk(0, pb0, sp0)
        fire_pk(1, pb1, sp1)
        fire_pk(2, pb2, sp2)
        fire_pk(3, pb3, sp3)
        wait_pk(0, pb0, sp0)
        fire_g(pb0, rb0, sg0)
        wait_pk(1, pb1, sp1)
        fire_g(pb1, rb1, sg1)

        @pl.loop(0, JCHUNKS, step=4)
        def _(j):
            slot(j + 0, pb0, sp0, pb2, sp2, rb0, sg0)
            slot(j + 1, pb1, sp1, pb3, sp3, rb1, sg1)
            slot(j + 2, pb2, sp2, pb0, sp0, rb0, sg0)
            slot(j + 3, pb3, sp3, pb1, sp1, rb1, sg1)

        plsc.subcore_barrier()

        @pl.when(sid < NS - 1)
        def _():
            pltpu.sync_copy(acc.at[pl.ds(r0, S_FULL)],
                            out_h.at[cid].at[pl.ds(r0, S_FULL)])

        @pl.when(sid == NS - 1)
        def _():
            pltpu.sync_copy(acc.at[pl.ds(r0, S_LAST)],
                            out_h.at[cid].at[pl.ds(r0, S_LAST)])

    return k(table, packed, zeros)


def _combine_body(p_ref, o_ref):
    o_ref[...] = p_ref[0] + p_ref[1]


def _combine(partial):
    return pl.pallas_call(
        _combine_body,
        grid=(N // BC,),
        in_specs=[pl.BlockSpec((NC, BC, D), lambda i: (0, i, 0))],
        out_specs=pl.BlockSpec((BC, D), lambda i: (i, 0)),
        out_shape=jax.ShapeDtypeStruct((N, D), jnp.float32),
    )(partial)


def kernel(feature, edge_index, edge_type, norm, weight):
    packed = _pack_edges(edge_index[0], edge_index[1], edge_type, norm[:, 0])
    table = _make_table(feature.astype(jnp.bfloat16),
                        weight.astype(jnp.bfloat16)).reshape(R * N, D)
    zeros = jnp.zeros((N, D), jnp.float32)
    partial = _sc_scatter(table, packed, zeros)
    return _combine(partial)


# speedup vs baseline: 3.7823x; 1.4722x over previous
import dataclasses
import functools

import jax
import jax.numpy as jnp
from jax import lax
from jax.experimental import pallas as pl
from jax.experimental.pallas import tpu as pltpu
from jax.experimental.pallas import tpu_sc as plsc

N = 10000
E = 320000
D = 128
R = 8

NC = 2
NS = 16
NW = NC * NS
CHUNK = 128
JCHUNKS = 80
EPW = CHUNK * JCHUNKS
EP = EPW * NW
NCHUNKS = EP // CHUNK
REC = 3 * CHUNK

S_FULL = 640
S_LAST = N - (NS - 1) * S_FULL

BN = 400
BP = 320
BC = 2000


def _mm_body(f_ref, w_ref, o_ref):
    o_ref[0] = jnp.dot(f_ref[...], w_ref[0], preferred_element_type=jnp.float32)


def _make_table(feature, weight):
    return pl.pallas_call(
        _mm_body,
        grid=(R,),
        in_specs=[
            pl.BlockSpec((N, D), lambda r: (0, 0)),
            pl.BlockSpec((1, D, D), lambda r: (r, 0, 0)),
        ],
        out_specs=pl.BlockSpec((1, N, D), lambda r: (r, 0, 0)),
        out_shape=jax.ShapeDtypeStruct((R, N, D), jnp.float32),
    )(feature, weight)


EC = E // CHUNK


def _prep_body(s_ref, e_ref, d_ref, n_ref, o_ref):
    o_ref[0:EC, 0, :] = e_ref[...] * N + s_ref[...]
    o_ref[0:EC, 1, :] = d_ref[...]
    o_ref[0:EC, 2, :] = lax.bitcast_convert_type(n_ref[...], jnp.int32)
    q = lax.broadcasted_iota(jnp.int32, (NCHUNKS - EC, CHUNK), 0)
    l = lax.broadcasted_iota(jnp.int32, (NCHUNKS - EC, CHUNK), 1)
    p = q * CHUNK + l
    o_ref[EC:NCHUNKS, 0, :] = (p % R) * N + p
    o_ref[EC:NCHUNKS, 1, :] = p
    o_ref[EC:NCHUNKS, 2, :] = jnp.zeros((NCHUNKS - EC, CHUNK), jnp.int32)


def _pack_edges(src, dst, et, nrm):
    packed = pl.pallas_call(
        _prep_body,
        out_shape=jax.ShapeDtypeStruct((NCHUNKS, 3, CHUNK), jnp.int32),
    )(src.reshape(EC, CHUNK), et.reshape(EC, CHUNK),
      dst.reshape(EC, CHUNK), nrm.reshape(EC, CHUNK))
    return packed.reshape(NCHUNKS * REC)


def _sc_compiler_params():
    cp = pltpu.CompilerParams()
    if "needs_layout_passes" in pltpu.CompilerParams.__dataclass_fields__:
        cp = dataclasses.replace(cp, needs_layout_passes=False)
    return cp


def _sc_scatter(table, packed, zeros):
    mesh = plsc.VectorSubcoreMesh(core_axis_name="c", subcore_axis_name="s")

    @functools.partial(
        pl.kernel,
        compiler_params=_sc_compiler_params(),
        out_type=jax.ShapeDtypeStruct((NC, N, D), jnp.float32),
        mesh=mesh,
        scratch_types=[
            pltpu.VMEM((REC,), jnp.int32),
            pltpu.VMEM((REC,), jnp.int32),
            pltpu.VMEM((REC,), jnp.int32),
            pltpu.VMEM((REC,), jnp.int32),
            pltpu.VMEM((CHUNK,), jnp.int32),
            pltpu.VMEM((CHUNK, D), jnp.float32),
            pltpu.VMEM((CHUNK, D), jnp.float32),
            pltpu.VMEM_SHARED((N, D), jnp.float32),
            pltpu.SemaphoreType.DMA,
            pltpu.SemaphoreType.DMA,
            pltpu.SemaphoreType.DMA,
            pltpu.SemaphoreType.DMA,
            pltpu.SemaphoreType.DMA,
            pltpu.SemaphoreType.DMA,
        ],
    )
    def k(table_h, packed_h, zeros_h, out_h,
          pb0, pb1, pb2, pb3, db, rb0, rb1, acc,
          sp0, sp1, sp2, sp3, sg0, sg1):
        cid = lax.axis_index("c")
        sid = lax.axis_index("s")
        wid = cid * NS + sid
        r0 = pl.multiple_of(sid * S_FULL, 128)

        @pl.when(sid < NS - 1)
        def _():
            pltpu.sync_copy(zeros_h.at[pl.ds(r0, S_FULL)],
                            acc.at[pl.ds(r0, S_FULL)])

        @pl.when(sid == NS - 1)
        def _():
            pltpu.sync_copy(zeros_h.at[pl.ds(r0, S_LAST)],
                            acc.at[pl.ds(r0, S_LAST)])

        plsc.subcore_barrier()

        def rec_off(j):
            return pl.multiple_of((wid * JCHUNKS + j) * REC, 128)

        def fire_pk(j, pb, sp):
            pltpu.make_async_copy(
                packed_h.at[pl.ds(rec_off(j), REC)], pb, sp).start()

        def wait_pk(j, pb, sp):
            pltpu.make_async_copy(
                packed_h.at[pl.ds(rec_off(j), REC)], pb, sp).wait()

        def fire_g(pb, rb, sg):
            pltpu.make_async_copy(
                table_h.at[pb.at[pl.ds(0, CHUNK)]], rb, sg).start()

        def wait_g(pb, rb, sg):
            pltpu.make_async_copy(
                table_h.at[pb.at[pl.ds(0, CHUNK)]], rb, sg).wait()

        def process(pb, rb):
            for m in range(CHUNK // 16):
                db[pl.ds(m * 16, 16)] = pb[pl.ds(CHUNK + m * 16, 16)]

            @pl.loop(0, CHUNK, step=16)
            def _(i):
                nv16 = plsc.bitcast(pb[pl.ds(2 * CHUNK + i, 16)], jnp.float32)
                for l in range(16):
                    s = nv16[l]
                    for c in range(D // 16):
                        rb[i + l, pl.ds(c * 16, 16)] = (
                            rb[i + l, pl.ds(c * 16, 16)] * s)

            pltpu.sync_copy(rb, acc.at[db], add=True)

        def slot(jj, pb_cur, sp_cur, pb_n2, sp_n2, rb, sg):
            wait_g(pb_cur, rb, sg)
            process(pb_cur, rb)

            @pl.when(jj + 4 < JCHUNKS)
            def _():
                fire_pk(jj + 4, pb_cur, sp_cur)

            @pl.when(jj + 2 < JCHUNKS)
            def _():
                wait_pk(jj + 2, pb_n2, sp_n2)
                fire_g(pb_n2, rb, sg)

        fire_pk(0, pb0, sp0)
        fire_pk(1, pb1, sp1)
        fire_pk(2, pb2, sp2)
        fire_pk(3, pb3, sp3)
        wait_pk(0, pb0, sp0)
        fire_g(pb0, rb0, sg0)
        wait_pk(1, pb1, sp1)
        fire_g(pb1, rb1, sg1)

        @pl.loop(0, JCHUNKS, step=4)
        def _(j):
            slot(j + 0, pb0, sp0, pb2, sp2, rb0, sg0)
            slot(j + 1, pb1, sp1, pb3, sp3, rb1, sg1)
            slot(j + 2, pb2, sp2, pb0, sp0, rb0, sg0)
            slot(j + 3, pb3, sp3, pb1, sp1, rb1, sg1)

        plsc.subcore_barrier()

        @pl.when(sid < NS - 1)
        def _():
            pltpu.sync_copy(acc.at[pl.ds(r0, S_FULL)],
                            out_h.at[cid].at[pl.ds(r0, S_FULL)])

        @pl.when(sid == NS - 1)
        def _():
            pltpu.sync_copy(acc.at[pl.ds(r0, S_LAST)],
                            out_h.at[cid].at[pl.ds(r0, S_LAST)])

    return k(table, packed, zeros)


def _combine_body(p_ref, o_ref):
    o_ref[...] = p_ref[0] + p_ref[1]


def _combine(partial):
    return pl.pallas_call(
        _combine_body,
        grid=(N // BC,),
        in_specs=[pl.BlockSpec((NC, BC, D), lambda i: (0, i, 0))],
        out_specs=pl.BlockSpec((BC, D), lambda i: (i, 0)),
        out_shape=jax.ShapeDtypeStruct((N, D), jnp.float32),
    )(partial)


def kernel(feature, edge_index, edge_type, norm, weight):
    packed = _pack_edges(edge_index[0], edge_index[1], edge_type, norm[:, 0])
    table = _make_table(feature.astype(jnp.bfloat16),
                        weight.astype(jnp.bfloat16)).reshape(R * N, D)
    zeros = jnp.zeros((N, D), jnp.float32)
    partial = _sc_scatter(table, packed, zeros)
    return _combine(partial)


# speedup vs baseline: 3.9226x; 1.0371x over previous
import dataclasses
import functools

import jax
import jax.numpy as jnp
from jax import lax
from jax.experimental import pallas as pl
from jax.experimental.pallas import tpu as pltpu
from jax.experimental.pallas import tpu_sc as plsc

N = 10000
E = 320000
D = 128
R = 8

NC = 2
NS = 16
NW = NC * NS
CHUNK = 128
JCHUNKS = 80
EPW = CHUNK * JCHUNKS
EP = EPW * NW
NCHUNKS = EP // CHUNK
REC = 3 * CHUNK

S_FULL = 640
S_LAST = N - (NS - 1) * S_FULL

BN = 400
BP = 320
BC = 2000


def _mm_body(f_ref, w_ref, o_ref):
    o_ref[0] = jnp.dot(f_ref[...], w_ref[0], preferred_element_type=jnp.float32)


def _make_table(feature, weight):
    return pl.pallas_call(
        _mm_body,
        grid=(R,),
        in_specs=[
            pl.BlockSpec((N, D), lambda r: (0, 0)),
            pl.BlockSpec((1, D, D), lambda r: (r, 0, 0)),
        ],
        out_specs=pl.BlockSpec((1, N, D), lambda r: (r, 0, 0)),
        out_shape=jax.ShapeDtypeStruct((R, N, D), jnp.float32),
    )(feature, weight)


EC = E // CHUNK


def _prep_body(s_ref, e_ref, d_ref, n_ref, o_ref):
    o_ref[0:EC, 0, :] = e_ref[...] * N + s_ref[...]
    o_ref[0:EC, 1, :] = d_ref[...]
    o_ref[0:EC, 2, :] = lax.bitcast_convert_type(n_ref[...], jnp.int32)
    q = lax.broadcasted_iota(jnp.int32, (NCHUNKS - EC, CHUNK), 0)
    l = lax.broadcasted_iota(jnp.int32, (NCHUNKS - EC, CHUNK), 1)
    p = q * CHUNK + l
    o_ref[EC:NCHUNKS, 0, :] = (p % R) * N + p
    o_ref[EC:NCHUNKS, 1, :] = p
    o_ref[EC:NCHUNKS, 2, :] = jnp.zeros((NCHUNKS - EC, CHUNK), jnp.int32)


def _pack_edges(src, dst, et, nrm):
    packed = pl.pallas_call(
        _prep_body,
        out_shape=jax.ShapeDtypeStruct((NCHUNKS, 3, CHUNK), jnp.int32),
    )(src.reshape(EC, CHUNK), et.reshape(EC, CHUNK),
      dst.reshape(EC, CHUNK), nrm.reshape(EC, CHUNK))
    return packed


def _sc_compiler_params():
    cp = pltpu.CompilerParams()
    if "needs_layout_passes" in pltpu.CompilerParams.__dataclass_fields__:
        cp = dataclasses.replace(cp, needs_layout_passes=False)
    return cp


def _sc_scatter(table, packed, zeros):
    mesh = plsc.VectorSubcoreMesh(core_axis_name="c", subcore_axis_name="s")

    @functools.partial(
        pl.kernel,
        compiler_params=_sc_compiler_params(),
        out_type=jax.ShapeDtypeStruct((NC, N, D), jnp.float32),
        mesh=mesh,
        scratch_types=[
            pltpu.VMEM((3, CHUNK), jnp.int32),
            pltpu.VMEM((3, CHUNK), jnp.int32),
            pltpu.VMEM((3, CHUNK), jnp.int32),
            pltpu.VMEM((3, CHUNK), jnp.int32),
            pltpu.VMEM((CHUNK,), jnp.int32),
            pltpu.VMEM((CHUNK, D), jnp.float32),
            pltpu.VMEM((CHUNK, D), jnp.float32),
            pltpu.VMEM_SHARED((N, D), jnp.float32),
            pltpu.SemaphoreType.DMA,
            pltpu.SemaphoreType.DMA,
            pltpu.SemaphoreType.DMA,
            pltpu.SemaphoreType.DMA,
            pltpu.SemaphoreType.DMA,
            pltpu.SemaphoreType.DMA,
        ],
    )
    def k(table_h, packed_h, zeros_h, out_h,
          pb0, pb1, pb2, pb3, db, rb0, rb1, acc,
          sp0, sp1, sp2, sp3, sg0, sg1):
        cid = lax.axis_index("c")
        sid = lax.axis_index("s")
        wid = cid * NS + sid
        r0 = pl.multiple_of(sid * S_FULL, 128)

        @pl.when(sid < NS - 1)
        def _():
            pltpu.sync_copy(zeros_h.at[pl.ds(r0, S_FULL)],
                            acc.at[pl.ds(r0, S_FULL)])

        @pl.when(sid == NS - 1)
        def _():
            pltpu.sync_copy(zeros_h.at[pl.ds(r0, S_LAST)],
                            acc.at[pl.ds(r0, S_LAST)])

        plsc.subcore_barrier()

        def fire_pk(j, pb, sp):
            pltpu.make_async_copy(
                packed_h.at[wid * JCHUNKS + j], pb, sp).start()

        def wait_pk(j, pb, sp):
            pltpu.make_async_copy(
                packed_h.at[wid * JCHUNKS + j], pb, sp).wait()

        def fire_g(pb, rb, sg):
            pltpu.make_async_copy(
                table_h.at[pb.at[0]], rb, sg).start()

        def wait_g(pb, rb, sg):
            pltpu.make_async_copy(
                table_h.at[pb.at[0]], rb, sg).wait()

        def process(pb, rb):
            for m in range(CHUNK // 16):
                db[pl.ds(m * 16, 16)] = pb[1, pl.ds(m * 16, 16)]

            @pl.loop(0, CHUNK, step=16)
            def _(i):
                nv16 = plsc.bitcast(pb[2, pl.ds(i, 16)], jnp.float32)
                for l in range(16):
                    s = nv16[l]
                    for c in range(D // 16):
                        rb[i + l, pl.ds(c * 16, 16)] = (
                            rb[i + l, pl.ds(c * 16, 16)] * s)

            pltpu.sync_copy(rb, acc.at[db], add=True)

        def slot(jj, pb_cur, sp_cur, pb_n2, sp_n2, rb, sg):
            wait_g(pb_cur, rb, sg)
            process(pb_cur, rb)

            @pl.when(jj + 4 < JCHUNKS)
            def _():
                fire_pk(jj + 4, pb_cur, sp_cur)

            @pl.when(jj + 2 < JCHUNKS)
            def _():
                wait_pk(jj + 2, pb_n2, sp_n2)
                fire_g(pb_n2, rb, sg)

        fire_pk(0, pb0, sp0)
        fire_pk(1, pb1, sp1)
        fire_pk(2, pb2, sp2)
        fire_pk(3, pb3, sp3)
        wait_pk(0, pb0, sp0)
        fire_g(pb0, rb0, sg0)
        wait_pk(1, pb1, sp1)
        fire_g(pb1, rb1, sg1)

        @pl.loop(0, JCHUNKS, step=4)
        def _(j):
            slot(j + 0, pb0, sp0, pb2, sp2, rb0, sg0)
            slot(j + 1, pb1, sp1, pb3, sp3, rb1, sg1)
            slot(j + 2, pb2, sp2, pb0, sp0, rb0, sg0)
            slot(j + 3, pb3, sp3, pb1, sp1, rb1, sg1)

        plsc.subcore_barrier()

        @pl.when(sid < NS - 1)
        def _():
            pltpu.sync_copy(acc.at[pl.ds(r0, S_FULL)],
                            out_h.at[cid].at[pl.ds(r0, S_FULL)])

        @pl.when(sid == NS - 1)
        def _():
            pltpu.sync_copy(acc.at[pl.ds(r0, S_LAST)],
                            out_h.at[cid].at[pl.ds(r0, S_LAST)])

    return k(table, packed, zeros)


def _combine_body(p_ref, o_ref):
    o_ref[...] = p_ref[0] + p_ref[1]


def _combine(partial):
    return pl.pallas_call(
        _combine_body,
        grid=(N // BC,),
        in_specs=[pl.BlockSpec((NC, BC, D), lambda i: (0, i, 0))],
        out_specs=pl.BlockSpec((BC, D), lambda i: (i, 0)),
        out_shape=jax.ShapeDtypeStruct((N, D), jnp.float32),
    )(partial)


def kernel(feature, edge_index, edge_type, norm, weight):
    packed = _pack_edges(edge_index[0], edge_index[1], edge_type, norm[:, 0])
    table = _make_table(feature.astype(jnp.bfloat16),
                        weight.astype(jnp.bfloat16)).reshape(R * N, D)
    zeros = jnp.zeros((N, D), jnp.float32)
    partial = _sc_scatter(table, packed, zeros)
    return _combine(partial)
